# baseline (speedup 1.0000x reference)
import functools

import jax
import jax.numpy as jnp
from jax.experimental import pallas as pl
from jax.experimental.pallas import tpu as pltpu


def _fused_kernel(ids_ref, table_ref, wpad_ref, cb_ref, wp_ref, bp_ref,
                  wg_ref, bg_ref, out_ref, oh_scr, u_scr, *, tt, maxw, lseg):
    nb = out_ref.shape[0]
    v, e = table_ref.shape
    pt = tt - 1
    i = pl.program_id(0)
    w = pl.program_id(1)

    @pl.when(jnp.logical_and(i == 0, w == 0))
    def _build_u():
        u_scr[...] = jnp.zeros(u_scr.shape, jnp.bfloat16)
        table = table_ref[...]
        for t in range(tt):
            for c in range(maxw):
                wslice = wpad_ref[pl.ds(c * e - t + pt, e), :]
                u_scr[t, c * lseg:c * lseg + v, :] = jnp.dot(
                    table, wslice, preferred_element_type=jnp.float32
                ).astype(jnp.bfloat16)

    @pl.when(w == 0)
    def _build_oh():
        iota = jax.lax.broadcasted_iota(jnp.int32, (nb, lseg), 1)
        for c in range(maxw):
            idc = ids_ref[0, c].astype(jnp.int32)
            oh_scr[:, c * lseg:(c + 1) * lseg] = (
                idc == iota).astype(jnp.bfloat16)

    @pl.when(w == 1)
    def _conv_highway():
        cb = cb_ref[0][None, :]
        mx = None
        for t in range(tt):
            acc = jnp.dot(oh_scr[...], u_scr[t],
                          preferred_element_type=jnp.float32)
            z = jnp.maximum(acc + cb, 0.0)
            mx = z if mx is None else jnp.maximum(mx, z)
        cnn = mx
        proj = jnp.maximum(
            jnp.dot(cnn, wp_ref[...], preferred_element_type=jnp.float32)
            + bp_ref[0][None, :], 0.0)
        gate = jax.nn.sigmoid(
            jnp.dot(cnn, wg_ref[...], preferred_element_type=jnp.float32)
            + bg_ref[0][None, :])
        out_ref[...] = gate * proj + (1.0 - gate) * cnn


def kernel(input_tensor, emb_table, conv_w, conv_b, w_proj, b_proj,
           w_gate, b_gate):
    s, b, maxw = input_tensor.shape
    v, e = emb_table.shape
    kk = conv_w.shape[2]
    n = s * b
    tt = maxw - kk + 1
    pt = tt - 1
    je = maxw * e
    lseg = 128

    nb = 2048
    nblocks = n // nb
    ids4 = (input_tensor.astype(jnp.int8).reshape(nblocks, nb, maxw)
            .transpose(0, 2, 1)[..., None])
    table_b = emb_table.astype(jnp.bfloat16)

    wlin = (jnp.pad(conv_w, ((0, 0), (0, 0), (0, maxw - kk)))
            .transpose(1, 2, 0).reshape(je, e))
    wpad = jnp.pad(wlin, ((pt, 0), (0, 0))).astype(jnp.bfloat16)

    cb2 = conv_b.reshape(1, e)
    bp2 = b_proj.reshape(1, e)
    bg2 = b_gate.reshape(1, e)
    wpt = w_proj.T
    wgt = w_gate.T

    out = pl.pallas_call(
        functools.partial(_fused_kernel, tt=tt, maxw=maxw, lseg=lseg),
        grid=(nblocks, 2),
        in_specs=[
            pl.BlockSpec((1, maxw, nb, 1), lambda i, w: (i, 0, 0, 0)),
            pl.BlockSpec((v, e), lambda i, w: (0, 0)),
            pl.BlockSpec((pt + je, e), lambda i, w: (0, 0)),
            pl.BlockSpec((1, e), lambda i, w: (0, 0)),
            pl.BlockSpec((e, e), lambda i, w: (0, 0)),
            pl.BlockSpec((1, e), lambda i, w: (0, 0)),
            pl.BlockSpec((e, e), lambda i, w: (0, 0)),
            pl.BlockSpec((1, e), lambda i, w: (0, 0)),
        ],
        out_specs=pl.BlockSpec((nb, e), lambda i, w: (i, 0)),
        out_shape=jax.ShapeDtypeStruct((n, e), jnp.float32),
        scratch_shapes=[
            pltpu.VMEM((nb, maxw * lseg), jnp.bfloat16),
            pltpu.VMEM((tt, maxw * lseg, e), jnp.bfloat16),
        ],
    )(ids4, table_b, wpad, cb2, wpt, bp2, wgt, bg2)

    return out.reshape(s, b, e)


# speedup vs baseline: 1.2148x; 1.2148x over previous
import functools

import jax
import jax.numpy as jnp
from jax.experimental import pallas as pl
from jax.experimental.pallas import tpu as pltpu


def _fused_kernel(ids_ref, table_ref, wpad_ref, cb_ref, wp_ref, bp_ref,
                  wg_ref, bg_ref, out_ref, oh_scr, u_scr, *, tt, maxw, lseg):
    nb = out_ref.shape[0]
    v, e = table_ref.shape
    pt = tt - 1
    i = pl.program_id(0)
    w = pl.program_id(1)

    @pl.when(jnp.logical_and(i == 0, w == 0))
    def _build_u():
        u_scr[...] = jnp.zeros(u_scr.shape, jnp.bfloat16)
        table = table_ref[...]
        for t in range(tt):
            for c in range(maxw):
                wslice = wpad_ref[pl.ds(c * e - t + pt, e), :]
                u_scr[t, c * lseg:c * lseg + v, :] = jnp.dot(
                    table, wslice, preferred_element_type=jnp.float32
                ).astype(jnp.bfloat16)

    @pl.when(w == 0)
    def _build_oh():
        iota = jax.lax.broadcasted_iota(jnp.int32, (nb, lseg), 1)
        for c in range(maxw):
            idc = ids_ref[0, c].astype(jnp.int32)
            oh_scr[:, c * lseg:(c + 1) * lseg] = (
                idc == iota).astype(jnp.bfloat16)

    @pl.when(w == 1)
    def _conv_highway():
        cb = cb_ref[0][None, :]
        mx = None
        for t in range(tt):
            acc = jnp.dot(oh_scr[...], u_scr[t],
                          preferred_element_type=jnp.float32)
            z = jnp.maximum(acc + cb, 0.0)
            mx = z if mx is None else jnp.maximum(mx, z)
        cnn = mx
        proj = jnp.maximum(
            jnp.dot(cnn, wp_ref[...], preferred_element_type=jnp.float32)
            + bp_ref[0][None, :], 0.0)
        gate = jax.nn.sigmoid(
            jnp.dot(cnn, wg_ref[...], preferred_element_type=jnp.float32)
            + bg_ref[0][None, :])
        out_ref[...] = gate * proj + (1.0 - gate) * cnn


def kernel(input_tensor, emb_table, conv_w, conv_b, w_proj, b_proj,
           w_gate, b_gate):
    s, b, maxw = input_tensor.shape
    v, e = emb_table.shape
    kk = conv_w.shape[2]
    n = s * b
    tt = maxw - kk + 1
    pt = tt - 1
    je = maxw * e
    lseg = 128

    nb = 512
    nblocks = n // nb
    ids4 = (input_tensor.astype(jnp.int8).reshape(nblocks, nb, maxw)
            .transpose(0, 2, 1)[..., None])
    table_b = emb_table.astype(jnp.bfloat16)

    wlin = (jnp.pad(conv_w, ((0, 0), (0, 0), (0, maxw - kk)))
            .transpose(1, 2, 0).reshape(je, e))
    wpad = jnp.pad(wlin, ((pt, 0), (0, 0))).astype(jnp.bfloat16)

    cb2 = conv_b.reshape(1, e)
    bp2 = b_proj.reshape(1, e)
    bg2 = b_gate.reshape(1, e)
    wpt = w_proj.T
    wgt = w_gate.T

    out = pl.pallas_call(
        functools.partial(_fused_kernel, tt=tt, maxw=maxw, lseg=lseg),
        grid=(nblocks, 2),
        in_specs=[
            pl.BlockSpec((1, maxw, nb, 1), lambda i, w: (i, 0, 0, 0)),
            pl.BlockSpec((v, e), lambda i, w: (0, 0)),
            pl.BlockSpec((pt + je, e), lambda i, w: (0, 0)),
            pl.BlockSpec((1, e), lambda i, w: (0, 0)),
            pl.BlockSpec((e, e), lambda i, w: (0, 0)),
            pl.BlockSpec((1, e), lambda i, w: (0, 0)),
            pl.BlockSpec((e, e), lambda i, w: (0, 0)),
            pl.BlockSpec((1, e), lambda i, w: (0, 0)),
        ],
        out_specs=pl.BlockSpec((nb, e), lambda i, w: (i, 0)),
        out_shape=jax.ShapeDtypeStruct((n, e), jnp.float32),
        scratch_shapes=[
            pltpu.VMEM((nb, maxw * lseg), jnp.bfloat16),
            pltpu.VMEM((tt, maxw * lseg, e), jnp.bfloat16),
        ],
    )(ids4, table_b, wpad, cb2, wpt, bp2, wgt, bg2)

    return out.reshape(s, b, e)


# speedup vs baseline: 1.2323x; 1.0144x over previous
import functools

import jax
import jax.numpy as jnp
from jax.experimental import pallas as pl
from jax.experimental.pallas import tpu as pltpu


def _fused_kernel(ids_ref, table_ref, wpad_ref, cb_ref, wp_ref, bp_ref,
                  wg_ref, bg_ref, out_ref, oh_scr, u_scr, *, tt, maxw, lseg):
    nb = out_ref.shape[0]
    v, e = table_ref.shape
    pt = tt - 1
    i = pl.program_id(0)
    w = pl.program_id(1)

    @pl.when(jnp.logical_and(i == 0, w == 0))
    def _build_u():
        u_scr[...] = jnp.zeros(u_scr.shape, jnp.bfloat16)
        table = table_ref[...]
        for t in range(tt):
            for c in range(maxw):
                wslice = wpad_ref[pl.ds(c * e - t + pt, e), :]
                u_scr[t, c * lseg:c * lseg + v, :] = jnp.dot(
                    table, wslice, preferred_element_type=jnp.float32
                ).astype(jnp.bfloat16)

    @pl.when(w == 0)
    def _build_oh():
        iota = jax.lax.broadcasted_iota(jnp.int32, (nb, lseg), 1)
        for c in range(maxw):
            idc = ids_ref[0, c].astype(jnp.int32)
            oh_scr[:, c * lseg:(c + 1) * lseg] = (
                idc == iota).astype(jnp.bfloat16)

    @pl.when(w == 1)
    def _conv_highway():
        cb = cb_ref[0][None, :]
        mx = None
        for t in range(tt):
            acc = jnp.dot(oh_scr[...], u_scr[t],
                          preferred_element_type=jnp.float32)
            z = jnp.maximum(acc + cb, 0.0)
            mx = z if mx is None else jnp.maximum(mx, z)
        cnn = mx
        proj = jnp.maximum(
            jnp.dot(cnn, wp_ref[...], preferred_element_type=jnp.float32)
            + bp_ref[0][None, :], 0.0)
        gate = jax.nn.sigmoid(
            jnp.dot(cnn, wg_ref[...], preferred_element_type=jnp.float32)
            + bg_ref[0][None, :])
        out_ref[...] = gate * proj + (1.0 - gate) * cnn


def kernel(input_tensor, emb_table, conv_w, conv_b, w_proj, b_proj,
           w_gate, b_gate):
    s, b, maxw = input_tensor.shape
    v, e = emb_table.shape
    kk = conv_w.shape[2]
    n = s * b
    tt = maxw - kk + 1
    pt = tt - 1
    je = maxw * e
    lseg = 128

    nb = 1024
    nblocks = n // nb
    ids4 = (input_tensor.astype(jnp.int8).reshape(nblocks, nb, maxw)
            .transpose(0, 2, 1)[..., None])
    table_b = emb_table.astype(jnp.bfloat16)

    wlin = (jnp.pad(conv_w, ((0, 0), (0, 0), (0, maxw - kk)))
            .transpose(1, 2, 0).reshape(je, e))
    wpad = jnp.pad(wlin, ((pt, 0), (0, 0))).astype(jnp.bfloat16)

    cb2 = conv_b.reshape(1, e)
    bp2 = b_proj.reshape(1, e)
    bg2 = b_gate.reshape(1, e)
    wpt = w_proj.T
    wgt = w_gate.T

    out = pl.pallas_call(
        functools.partial(_fused_kernel, tt=tt, maxw=maxw, lseg=lseg),
        grid=(nblocks, 2),
        in_specs=[
            pl.BlockSpec((1, maxw, nb, 1), lambda i, w: (i, 0, 0, 0)),
            pl.BlockSpec((v, e), lambda i, w: (0, 0)),
            pl.BlockSpec((pt + je, e), lambda i, w: (0, 0)),
            pl.BlockSpec((1, e), lambda i, w: (0, 0)),
            pl.BlockSpec((e, e), lambda i, w: (0, 0)),
            pl.BlockSpec((1, e), lambda i, w: (0, 0)),
            pl.BlockSpec((e, e), lambda i, w: (0, 0)),
            pl.BlockSpec((1, e), lambda i, w: (0, 0)),
        ],
        out_specs=pl.BlockSpec((nb, e), lambda i, w: (i, 0)),
        out_shape=jax.ShapeDtypeStruct((n, e), jnp.float32),
        scratch_shapes=[
            pltpu.VMEM((nb, maxw * lseg), jnp.bfloat16),
            pltpu.VMEM((tt, maxw * lseg, e), jnp.bfloat16),
        ],
    )(ids4, table_b, wpad, cb2, wpt, bp2, wgt, bg2)

    return out.reshape(s, b, e)
